# baseline (speedup 1.0000x reference)
import functools

import jax
import jax.numpy as jnp
from jax.experimental import pallas as pl
from jax.experimental.pallas import tpu as pltpu
from jax.experimental.pallas import tpu_sc as plsc

CHUNK = 2048
GWND = 128


def _sc_gather(table, idx, num):
    width = table.shape[1]
    mesh = plsc.VectorSubcoreMesh(core_axis_name="c", subcore_axis_name="s")

    @pl.kernel(out_type=jax.ShapeDtypeStruct((num, width), table.dtype),
               mesh=mesh)
    def k(x_hbm, i_hbm, o_hbm):
        def body(i_vmem, o_vmem):
            pltpu.sync_copy(x_hbm.at[i_vmem.at[0]], o_vmem)
        pltpu.emit_pipeline(
            body,
            grid=(num // GWND,),
            in_specs=[pl.BlockSpec((1, GWND), lambda i: (0, i))],
            out_specs=[pl.BlockSpec((GWND, width), lambda i: (i, 0))],
            core_axis_name=("c", "s"),
            dimension_semantics=(pltpu.PARALLEL,),
        )(i_hbm, o_hbm)

    return k(table, idx.reshape(1, num))


def _dot_bf16(a, b, dims=(((1,), (0,)), ((), ()))):
    return jax.lax.dot_general(a.astype(jnp.bfloat16), b.astype(jnp.bfloat16),
                               dimension_numbers=dims,
                               preferred_element_type=jnp.float32)


def _edge_mlp_scan_kernel(z_ref, xj_ref, d_ref, w1i_ref, w1j_ref, b1_ref,
                          w2_ref, b2_ref, w3_ref, b3_ref, out_ref,
                          carry_m, carry_d, carry_x):
    k = pl.program_id(0)

    @pl.when(k == 0)
    def _():
        carry_m[...] = jnp.zeros_like(carry_m)
        carry_d[...] = jnp.full_like(carry_d, -1)
        carry_x[...] = jnp.zeros_like(carry_x)

    d_row = d_ref[...].reshape(1, CHUNK)
    lane1 = jax.lax.broadcasted_iota(jnp.int32, (1, CHUNK), 1)
    cd_row = jnp.broadcast_to(carry_d[0:1, 0:1], (1, CHUNK))

    v = z_ref[...].T
    prev_d = jnp.where(lane1 == 0, cd_row, pltpu.roll(d_row, 1, 1))
    fl = (d_row != prev_d).astype(jnp.int32)
    s = 1
    while s < CHUNK:
        v_sh = pltpu.roll(v, s, 1)
        d_sh = jnp.where(lane1 < s, -1, pltpu.roll(d_row, s, 1))
        fl_sh = jnp.where(lane1 < s, 0, pltpu.roll(fl, s, 1))
        same = (d_row == d_sh) & (fl_sh == 1)
        take = same & (fl == 0)
        v = jnp.where(jnp.broadcast_to(take, (32, CHUNK)), v_sh, v)
        fl = jnp.maximum(fl, same.astype(jnp.int32))
        s *= 2
    in_carry = d_row == cd_row
    v = jnp.where(jnp.broadcast_to(in_carry, (32, CHUNK)),
                  jnp.broadcast_to(carry_x[:, 0:1], (32, CHUNK)), v)
    carry_x[:, 0:1] = v[:, CHUNK - 1:CHUNK]

    xi_t = v
    xd_t = xj_ref[...][:, :32].T - xi_t
    dims_t = (((0,), (0,)), ((), ()))
    h = _dot_bf16(xi_t, w1i_ref[...], dims_t) + _dot_bf16(xd_t, w1j_ref[...],
                                                          dims_t)
    h = jnp.maximum(h + b1_ref[...], 0.0)
    h = jnp.maximum(_dot_bf16(h, w2_ref[...]) + b2_ref[...], 0.0)
    m = _dot_bf16(w3_ref[...], h, (((0,), (1,)), ((), ())))
    m = m + b3_ref[...]

    d = jnp.broadcast_to(d_row, (8, CHUNK))

    lane = jax.lax.broadcasted_iota(jnp.int32, (8, CHUNK), 1)
    s = 1
    while s < CHUNK:
        m_sh = pltpu.roll(m, s, 1)
        d_sh = jnp.where(lane < s, -1, pltpu.roll(d, s, 1))
        m = jnp.where(d == d_sh, jnp.maximum(m, m_sh), m)
        s *= 2

    cm = jnp.broadcast_to(carry_m[:, 0:1], (8, CHUNK))
    cd = jnp.broadcast_to(carry_d[:, 0:1], (8, CHUNK))
    m = jnp.where(d == cd, jnp.maximum(m, cm), m)

    carry_m[:, 0:1] = m[:, CHUNK - 1:CHUNK]
    carry_d[:, 0:1] = d[:, CHUNK - 1:CHUNK]

    out_ref[...] = m


def _edge_mlp_scan(z, xj, d3, w1i, w1j, b1, w2, b2, w3, b3, n_chunks):
    epad = n_chunks * CHUNK
    return pl.pallas_call(
        _edge_mlp_scan_kernel,
        grid=(n_chunks,),
        in_specs=[
            pl.BlockSpec((CHUNK, 32), lambda i: (i, 0)),
            pl.BlockSpec((CHUNK, 128), lambda i: (i, 0)),
            pl.BlockSpec((1, 1, CHUNK), lambda i: (i, 0, 0)),
            pl.BlockSpec((32, 128), lambda i: (0, 0)),
            pl.BlockSpec((32, 128), lambda i: (0, 0)),
            pl.BlockSpec((1, 128), lambda i: (0, 0)),
            pl.BlockSpec((128, 128), lambda i: (0, 0)),
            pl.BlockSpec((1, 128), lambda i: (0, 0)),
            pl.BlockSpec((128, 8), lambda i: (0, 0)),
            pl.BlockSpec((8, 1), lambda i: (0, 0)),
        ],
        out_specs=pl.BlockSpec((8, CHUNK), lambda i: (0, i)),
        out_shape=jax.ShapeDtypeStruct((8, epad), jnp.float32),
        scratch_shapes=[
            pltpu.VMEM((8, 128), jnp.float32),
            pltpu.VMEM((8, 128), jnp.int32),
            pltpu.VMEM((32, 128), jnp.float32),
        ],
    )(z, xj, d3, w1i, w1j, b1, w2, b2, w3, b3)


def _prep_edges(edge_index, n):
    src = edge_index[0].astype(jnp.int32)
    dst = edge_index[1].astype(jnp.int32)
    e = src.shape[0]
    n_chunks = -(-e // CHUNK)
    epad = n_chunks * CHUNK
    perm = jnp.argsort(dst)
    dst_s = dst[perm]
    src_s = src[perm]
    dst_scan = jnp.concatenate([dst_s, jnp.full((epad - e,), n, jnp.int32)])
    dst_g = jnp.concatenate([dst_s, jnp.zeros((epad - e,), jnp.int32)])
    src_g = jnp.concatenate([src_s, jnp.zeros((epad - e,), jnp.int32)])
    counts = jax.ops.segment_sum(jnp.ones((e,), jnp.int32), dst,
                                 num_segments=n)
    end = jnp.cumsum(counts)
    has_edge = counts > 0
    last = jnp.maximum(end - 1, 0)
    d3 = dst_scan.reshape(n_chunks, 1, CHUNK)
    seed_pos = jnp.where(has_edge, end - counts, epad)
    return src_g, seed_pos, d3, last, has_edge, n_chunks


def _pad128(x):
    n, d = x.shape
    return jnp.concatenate([x, jnp.zeros((n, 128 - d), x.dtype)], axis=1)


def _edge_conv(x_feats, prep, layers):
    src_g, seed_pos, d3, last, has_edge, n_chunks = prep
    (w1, b1), (w2, b2), (w3, b3) = layers
    n, d = x_feats.shape
    epad = n_chunks * CHUNK
    w1i = jnp.zeros((32, 128), jnp.float32).at[:d].set(w1[:d])
    w1j = jnp.zeros((32, 128), jnp.float32).at[:d].set(w1[d:])
    x32 = jnp.concatenate([x_feats, jnp.zeros((n, 32 - d), jnp.float32)],
                          axis=1)
    z = jnp.zeros((epad, 32), jnp.float32).at[seed_pos].set(
        x32, mode='drop', unique_indices=True)
    xj = _sc_gather(_pad128(x_feats), src_g, epad)
    sc = _edge_mlp_scan(z, xj, d3, w1i, w1j, b1[None, :], w2,
                        b2[None, :], w3, b3[:, None], n_chunks)
    return jnp.where(has_edge[:, None], sc[:, last].T, 0.0)


def _batch_norm(x, gamma, beta, eps=1e-5):
    mean = jnp.mean(x, axis=0)
    var = jnp.var(x, axis=0)
    return gamma * (x - mean) / jnp.sqrt(var + eps) + beta


def _mlp(x, layers):
    n = len(layers)
    for i, (w, b) in enumerate(layers):
        x = x @ w + b
        if i < n - 1:
            x = jax.nn.relu(x)
    return x


@jax.jit
def kernel(x0, edge_index0, x1, edge_index1, clusters0, params):
    n0 = x0.shape[0]
    n1 = x1.shape[0]
    prep0 = _prep_edges(edge_index0, n0)
    prep1 = _prep_edges(edge_index1, n1)

    h0 = _edge_conv(x0, prep0, params['Lconv0'])
    h0 = jax.nn.relu(h0)
    h0 = _batch_norm(h0, params['Lnorm0'][0], params['Lnorm0'][1])

    cl = clusters0.astype(jnp.int32)
    sums = jax.ops.segment_sum(h0, cl, num_segments=n1)
    cnt = jax.ops.segment_sum(jnp.ones((n0,), jnp.float32), cl,
                              num_segments=n1)
    p1 = sums / jnp.maximum(cnt, 1.0)[:, None]

    f1 = jnp.concatenate([x1[:, :2], p1], axis=1)
    X = _edge_conv(f1, prep1, params['Lconv1'])
    X = jax.nn.relu(X)
    X = _batch_norm(X, params['Lnorm1'][0], params['Lnorm1'][1])

    f2 = jnp.concatenate([x1[:, :2], p1, X], axis=1)
    X = _edge_conv(f2, prep1, params['Rconv1'])
    X = jax.nn.relu(X)
    X = _batch_norm(X, params['Rnorm1'][0], params['Rnorm1'][1])

    Xup = X[cl]
    f3 = jnp.concatenate([x0[:, :2], h0, Xup], axis=1)
    X = _edge_conv(f3, prep0, params['Rconv0'])
    X = jax.nn.relu(X)
    X = _batch_norm(X, params['Rnorm0'][0], params['Rnorm0'][1])

    return _mlp(X, params['mlp_out'])


# speedup vs baseline: 1.4934x; 1.4934x over previous
import jax
import jax.numpy as jnp
from jax.experimental import pallas as pl
from jax.experimental.pallas import tpu as pltpu
from jax.experimental.pallas import tpu_sc as plsc

CHUNK = 2048
GWND = 128


def _sc_gather(table, idx, num):
    width = table.shape[1]
    mesh = plsc.VectorSubcoreMesh(core_axis_name="c", subcore_axis_name="s")

    @pl.kernel(out_type=jax.ShapeDtypeStruct((num, width), table.dtype),
               mesh=mesh)
    def k(x_hbm, i_hbm, o_hbm):
        def body(i_vmem, o_vmem):
            pltpu.sync_copy(x_hbm.at[i_vmem.at[0]], o_vmem)
        pltpu.emit_pipeline(
            body,
            grid=(num // GWND,),
            in_specs=[pl.BlockSpec((1, GWND), lambda i: (0, i))],
            out_specs=[pl.BlockSpec((GWND, width), lambda i: (i, 0))],
            core_axis_name=("c", "s"),
            dimension_semantics=(pltpu.PARALLEL,),
        )(i_hbm, o_hbm)

    return k(table, idx.reshape(1, num))


def _dot_bf16(a, b, dims=(((1,), (0,)), ((), ()))):
    return jax.lax.dot_general(a.astype(jnp.bfloat16), b.astype(jnp.bfloat16),
                               dimension_numbers=dims,
                               preferred_element_type=jnp.float32)


def _edge_mlp_scan_kernel(xi_ref, xj_ref, d_ref, w1i_ref, w1j_ref, b1_ref,
                          w2_ref, b2_ref, w3_ref, b3_ref, out_ref,
                          carry_m, carry_d):
    k = pl.program_id(0)

    @pl.when(k == 0)
    def _():
        carry_m[...] = jnp.zeros_like(carry_m)
        carry_d[...] = jnp.full_like(carry_d, -1)

    xi = xi_ref[...]
    xd = xj_ref[...] - xi
    h = _dot_bf16(xi, w1i_ref[...]) + _dot_bf16(xd, w1j_ref[...])
    h = jnp.maximum(h + b1_ref[...], 0.0)
    h = jnp.maximum(_dot_bf16(h, w2_ref[...]) + b2_ref[...], 0.0)
    m = _dot_bf16(w3_ref[...], h, (((0,), (1,)), ((), ())))
    m = m + b3_ref[...]

    d = jnp.broadcast_to(d_ref[...].reshape(1, CHUNK), (8, CHUNK))

    lane = jax.lax.broadcasted_iota(jnp.int32, (8, CHUNK), 1)
    s = 1
    while s < CHUNK:
        m_sh = pltpu.roll(m, s, 1)
        d_sh = jnp.where(lane < s, -1, pltpu.roll(d, s, 1))
        m = jnp.where(d == d_sh, jnp.maximum(m, m_sh), m)
        s *= 2

    cm = jnp.broadcast_to(carry_m[:, 0:1], (8, CHUNK))
    cd = jnp.broadcast_to(carry_d[:, 0:1], (8, CHUNK))
    m = jnp.where(d == cd, jnp.maximum(m, cm), m)

    carry_m[:, 0:1] = m[:, CHUNK - 1:CHUNK]
    carry_d[:, 0:1] = d[:, CHUNK - 1:CHUNK]

    out_ref[...] = m


def _edge_mlp_scan(xi, xj, d3, w1i, w1j, b1, w2, b2, w3, b3, n_chunks):
    epad = n_chunks * CHUNK
    return pl.pallas_call(
        _edge_mlp_scan_kernel,
        grid=(n_chunks,),
        in_specs=[
            pl.BlockSpec((CHUNK, 128), lambda i: (i, 0)),
            pl.BlockSpec((CHUNK, 128), lambda i: (i, 0)),
            pl.BlockSpec((1, 1, CHUNK), lambda i: (i, 0, 0)),
            pl.BlockSpec((128, 128), lambda i: (0, 0)),
            pl.BlockSpec((128, 128), lambda i: (0, 0)),
            pl.BlockSpec((1, 128), lambda i: (0, 0)),
            pl.BlockSpec((128, 128), lambda i: (0, 0)),
            pl.BlockSpec((1, 128), lambda i: (0, 0)),
            pl.BlockSpec((128, 8), lambda i: (0, 0)),
            pl.BlockSpec((8, 1), lambda i: (0, 0)),
        ],
        out_specs=pl.BlockSpec((8, CHUNK), lambda i: (0, i)),
        out_shape=jax.ShapeDtypeStruct((8, epad), jnp.float32),
        scratch_shapes=[
            pltpu.VMEM((8, 128), jnp.float32),
            pltpu.VMEM((8, 128), jnp.int32),
        ],
    )(xi, xj, d3, w1i, w1j, b1, w2, b2, w3, b3)


def _prep_edges(edge_index, n):
    src = edge_index[0].astype(jnp.int32)
    dst = edge_index[1].astype(jnp.int32)
    e = src.shape[0]
    n_chunks = -(-e // CHUNK)
    epad = n_chunks * CHUNK
    perm = jnp.argsort(dst)
    dst_s = dst[perm]
    src_s = src[perm]
    dst_scan = jnp.concatenate([dst_s, jnp.full((epad - e,), n, jnp.int32)])
    dst_g = jnp.concatenate([dst_s, jnp.zeros((epad - e,), jnp.int32)])
    src_g = jnp.concatenate([src_s, jnp.zeros((epad - e,), jnp.int32)])
    counts = jax.ops.segment_sum(jnp.ones((e,), jnp.int32), dst,
                                 num_segments=n)
    end = jnp.cumsum(counts)
    has_edge = counts > 0
    last = jnp.maximum(end - 1, 0)
    d3 = dst_scan.reshape(n_chunks, 1, CHUNK)
    return src_g, dst_g, d3, last, has_edge, n_chunks


def _pad128(x):
    n, d = x.shape
    return jnp.concatenate([x, jnp.zeros((n, 128 - d), x.dtype)], axis=1)


def _edge_conv(x_feats, prep, layers):
    src_g, dst_g, d3, last, has_edge, n_chunks = prep
    (w1, b1), (w2, b2), (w3, b3) = layers
    d = x_feats.shape[1]
    epad = n_chunks * CHUNK
    w1i = jnp.zeros((128, 128), jnp.float32).at[:d].set(w1[:d])
    w1j = jnp.zeros((128, 128), jnp.float32).at[:d].set(w1[d:])
    xp = _pad128(x_feats)
    xi = _sc_gather(xp, dst_g, epad)
    xj = _sc_gather(xp, src_g, epad)
    sc = _edge_mlp_scan(xi, xj, d3, w1i, w1j, b1[None, :], w2,
                        b2[None, :], w3, b3[:, None], n_chunks)
    return jnp.where(has_edge[:, None], sc[:, last].T, 0.0)


def _batch_norm(x, gamma, beta, eps=1e-5):
    mean = jnp.mean(x, axis=0)
    var = jnp.var(x, axis=0)
    return gamma * (x - mean) / jnp.sqrt(var + eps) + beta


def _mlp(x, layers):
    n = len(layers)
    for i, (w, b) in enumerate(layers):
        x = x @ w + b
        if i < n - 1:
            x = jax.nn.relu(x)
    return x


@jax.jit
def kernel(x0, edge_index0, x1, edge_index1, clusters0, params):
    n0 = x0.shape[0]
    n1 = x1.shape[0]
    prep0 = _prep_edges(edge_index0, n0)
    prep1 = _prep_edges(edge_index1, n1)

    h0 = _edge_conv(x0, prep0, params['Lconv0'])
    h0 = jax.nn.relu(h0)
    h0 = _batch_norm(h0, params['Lnorm0'][0], params['Lnorm0'][1])

    cl = clusters0.astype(jnp.int32)
    sums = jax.ops.segment_sum(h0, cl, num_segments=n1)
    cnt = jax.ops.segment_sum(jnp.ones((n0,), jnp.float32), cl,
                              num_segments=n1)
    p1 = sums / jnp.maximum(cnt, 1.0)[:, None]

    f1 = jnp.concatenate([x1[:, :2], p1], axis=1)
    X = _edge_conv(f1, prep1, params['Lconv1'])
    X = jax.nn.relu(X)
    X = _batch_norm(X, params['Lnorm1'][0], params['Lnorm1'][1])

    f2 = jnp.concatenate([x1[:, :2], p1, X], axis=1)
    X = _edge_conv(f2, prep1, params['Rconv1'])
    X = jax.nn.relu(X)
    X = _batch_norm(X, params['Rnorm1'][0], params['Rnorm1'][1])

    Xup = X[cl]
    f3 = jnp.concatenate([x0[:, :2], h0, Xup], axis=1)
    X = _edge_conv(f3, prep0, params['Rconv0'])
    X = jax.nn.relu(X)
    X = _batch_norm(X, params['Rnorm0'][0], params['Rnorm0'][1])

    return _mlp(X, params['mlp_out'])
